# baseline (speedup 1.0000x reference)
import jax
import jax.numpy as jnp
from jax.experimental import pallas as pl
from jax.experimental.pallas import tpu as pltpu

_INPUT_DIM = 4096
_NUM_EXPERTS = 64
_TOP_K = 8
_ALPHA = 10.0
_N_TOKENS = 8192


_KSPLIT = 2


def _gate_body(x_ref, w_ref, b_ref, o_ref, acc_ref):
    kidx = pl.program_id(1)
    part = jax.lax.dot_general(
        x_ref[...], w_ref[...],
        dimension_numbers=(((1,), (1,)), ((), ())),
        preferred_element_type=jnp.float32)

    @pl.when(kidx == 0)
    def _init():
        acc_ref[...] = part

    @pl.when(kidx != 0)
    def _accum():
        acc_ref[...] += part

    @pl.when(kidx == _KSPLIT - 1)
    def _epilogue():
        _finish(acc_ref[...], b_ref, o_ref)


def _finish(logits, b_ref, o_ref):
    lt = logits.T + b_ref[...]
    neg_inf = jnp.float32(-jnp.inf)

    t = lt
    removed = jnp.zeros(lt.shape[1:], jnp.float32)[None, :]
    kth = jnp.full_like(removed, neg_inf)
    m0 = None
    for step in range(_TOP_K):
        m = jnp.max(t, axis=0, keepdims=True)
        if step == 0:
            m0 = m
        eq = t == m
        cnt = jnp.sum(jnp.where(eq, 1.0, 0.0), axis=0, keepdims=True)
        total = removed + cnt
        hit = jnp.logical_and(removed < float(_TOP_K),
                              total >= float(_TOP_K))
        kth = jnp.where(hit, m, kth)
        removed = total
        if step < _TOP_K - 1:
            t = jnp.where(eq, neg_inf, t)

    mask = lt < kth

    e0 = jnp.exp(lt - m0)
    inv_s = 1.0 / jnp.sum(e0, axis=0, keepdims=True)
    sm = e0 * inv_s

    m1 = _ALPHA * (jnp.exp(inv_s) - 1.0)
    p = 1.0 + sm
    p2 = p * p
    p4 = p2 * p2
    p10 = p4 * p4 * p2
    e1 = jnp.where(mask,
                   p10 * jnp.exp(-m1),
                   jnp.exp(_ALPHA * jnp.exp(sm) - (_ALPHA + m1)))
    gt = e1 * (1.0 / jnp.sum(e1, axis=0, keepdims=True))
    o_ref[...] = gt.T


@jax.jit
def kernel(x, W_gate, b_gate):
    b2 = b_gate.reshape(_NUM_EXPERTS, 1)
    tm = 1024
    kc = _INPUT_DIM // _KSPLIT
    grid = (_N_TOKENS // tm, _KSPLIT)
    return pl.pallas_call(
        _gate_body,
        grid=grid,
        in_specs=[
            pl.BlockSpec((tm, kc), lambda i, k: (i, k)),
            pl.BlockSpec((_NUM_EXPERTS, kc), lambda i, k: (0, k)),
            pl.BlockSpec((_NUM_EXPERTS, 1), lambda i, k: (0, 0)),
        ],
        out_specs=pl.BlockSpec((tm, _NUM_EXPERTS), lambda i, k: (i, 0)),
        out_shape=jax.ShapeDtypeStruct((_N_TOKENS, _NUM_EXPERTS),
                                       jnp.float32),
        scratch_shapes=[pltpu.VMEM((tm, _NUM_EXPERTS), jnp.float32)],
    )(x, W_gate, b2)
